# baseline (speedup 1.0000x reference)
import functools

import jax
import jax.numpy as jnp
from jax import lax
from jax.experimental import pallas as pl
from jax.experimental.pallas import tpu as pltpu
from jax.experimental.pallas import tpu_sc as plsc

EPS = 1e-05
LN_EPS = 1e-05
CAPACITY_FACTOR = 1.0


def kernel(hidden_states, ln_scale, ln_bias, gate_w, gate_proj_w, up_proj_w, down_proj_w):
    B, S, H = hidden_states.shape
    hid = hidden_states.reshape(-1, H)
    N = hid.shape[0]
    E = gate_w.shape[0]
    I = gate_proj_w.shape[1]
    C = int(N * CAPACITY_FACTOR / E)
    C = max(C, 1)
    C = min(C, N)

    def _router_kernel(x_ref, gw_ref, scale_ref, bias_ref,
                       idx_ref, prob_ref, aux_ref):
        x = x_ref[...]
        mean = jnp.mean(x, axis=1, keepdims=True)
        xc = x - mean
        var = jnp.mean(xc * xc, axis=1, keepdims=True)
        xn = xc * jax.lax.rsqrt(var + LN_EPS) * scale_ref[...] + bias_ref[...]
        logits = jax.lax.dot_general(
            gw_ref[...], xn, (((1,), (1,)), ((), ())),
            preferred_element_type=jnp.float32)

        tokmax = jnp.max(logits, axis=1, keepdims=True)
        ex = jnp.exp(logits - tokmax)
        denom = jnp.sum(ex, axis=1, keepdims=True)
        pfull = ex / denom

        expmax = jnp.max(logits, axis=0, keepdims=True)
        lse = jnp.log(jnp.sum(jnp.exp(logits - expmax), axis=0,
                              keepdims=True)) + expmax
        aux_ref[...] = jnp.full((1, 1), 0.001, jnp.float32) * jnp.mean(lse * lse)

        iota_c = jax.lax.broadcasted_iota(jnp.int32, (E, N), 1)
        work = pfull
        idx_cols = []
        val_cols = []
        for _ in range(C):
            cur = jnp.max(work, axis=1, keepdims=True)
            cand = jnp.where(work == cur, iota_c, jnp.int32(N))
            amin = jnp.min(cand, axis=1, keepdims=True)
            hit = iota_c == amin
            work = jnp.where(hit, -1.0, work)
            idx_cols.append(amin)
            val_cols.append(cur)
        idx_ref[...] = jnp.concatenate(idx_cols, axis=1)
        prob_ref[...] = jnp.concatenate(val_cols, axis=1)

    idx, prob, aux = pl.pallas_call(
        _router_kernel,
        out_shape=[
            jax.ShapeDtypeStruct((E, C), jnp.int32),
            jax.ShapeDtypeStruct((E, C), jnp.float32),
            jax.ShapeDtypeStruct((1, 1), jnp.float32),
        ],
    )(hid, gate_w, ln_scale.reshape(1, H), ln_bias.reshape(1, H))

    info = plsc.get_sparse_core_info()
    NW = info.num_cores * info.num_subcores
    rows_per_w = (E * C) // NW
    mesh = plsc.VectorSubcoreMesh(core_axis_name="c", subcore_axis_name="s")

    @functools.partial(
        pl.kernel, mesh=mesh,
        out_type=jax.ShapeDtypeStruct((E * C, H), jnp.float32),
        scratch_types=[
            pltpu.VMEM((rows_per_w,), jnp.int32),
            pltpu.VMEM((rows_per_w, H), jnp.float32),
            pltpu.SemaphoreType.DMA,
        ],
    )
    def _sc_gather(hid_hbm, idxf_hbm, out_hbm, idx_v, rows_v, sem):
        wid = lax.axis_index("s") * info.num_cores + lax.axis_index("c")
        base = wid * rows_per_w
        pltpu.sync_copy(idxf_hbm.at[pl.ds(base, rows_per_w)], idx_v)
        pltpu.async_copy(hid_hbm.at[idx_v], rows_v, sem).wait()
        pltpu.sync_copy(rows_v, out_hbm.at[pl.ds(base, rows_per_w)])

    xgath = _sc_gather(hid, idx.reshape(-1))

    NCH = 2
    I2 = I // NCH

    def _ffn_kernel(idx_ref, prob_ref, xin_ref, gp_ref, up_ref, dp_ref,
                    out_ref, cnt_ref, acc_ref):
        e = pl.program_id(0)
        i = pl.program_id(1)

        @pl.when(jnp.logical_and(e == 0, i == 0))
        def _():
            out_ref[...] = jnp.zeros_like(out_ref)
            cnt_ref[...] = jnp.zeros_like(cnt_ref)

        x = xin_ref[...]
        g = jax.lax.dot_general(x, gp_ref[0], (((1,), (1,)), ((), ())),
                                preferred_element_type=jnp.float32)
        u = jax.lax.dot_general(x, up_ref[0], (((1,), (1,)), ((), ())),
                                preferred_element_type=jnp.float32)
        h = g * jax.nn.sigmoid(g) * u
        o = jax.lax.dot_general(h, dp_ref[0], (((1,), (1,)), ((), ())),
                                preferred_element_type=jnp.float32)

        @pl.when(i == 0)
        def _():
            acc_ref[...] = o

        @pl.when(i > 0)
        def _():
            acc_ref[...] = acc_ref[...] + o

        @pl.when(i == NCH - 1)
        def _():
            oe = acc_ref[...]
            for c in range(C):
                t = idx_ref[e, c]
                p = prob_ref[e, c]
                out_ref[t, :] = out_ref[t, :] + oe[c, :] * p
                cnt_ref[pl.ds(t, 1), :] = cnt_ref[pl.ds(t, 1), :] + p

        @pl.when(jnp.logical_and(e == pl.num_programs(0) - 1, i == NCH - 1))
        def _():
            out_ref[...] = out_ref[...] / jnp.maximum(cnt_ref[...], EPS)

    out = pl.pallas_call(
        _ffn_kernel,
        grid=(E, NCH),
        in_specs=[
            pl.BlockSpec(memory_space=pltpu.SMEM),
            pl.BlockSpec(memory_space=pltpu.SMEM),
            pl.BlockSpec((C, H), lambda e, i: (e, 0)),
            pl.BlockSpec((1, I2, H), lambda e, i: (e, i, 0)),
            pl.BlockSpec((1, I2, H), lambda e, i: (e, i, 0)),
            pl.BlockSpec((1, H, I2), lambda e, i: (e, 0, i)),
        ],
        out_specs=pl.BlockSpec((N, H), lambda e, i: (0, 0)),
        out_shape=jax.ShapeDtypeStruct((N, H), jnp.float32),
        scratch_shapes=[pltpu.VMEM((N, 1), jnp.float32),
                        pltpu.VMEM((C, H), jnp.float32)],
        compiler_params=pltpu.CompilerParams(
            dimension_semantics=("arbitrary", "arbitrary")),
    )(idx, prob, xgath.reshape(E * C, H), gate_proj_w, up_proj_w, down_proj_w)

    return out.reshape(B, S, H), aux.reshape(())


# speedup vs baseline: 1.0446x; 1.0446x over previous
import functools

import jax
import jax.numpy as jnp
from jax import lax
from jax.experimental import pallas as pl
from jax.experimental.pallas import tpu as pltpu
from jax.experimental.pallas import tpu_sc as plsc

EPS = 1e-05
LN_EPS = 1e-05
CAPACITY_FACTOR = 1.0


def kernel(hidden_states, ln_scale, ln_bias, gate_w, gate_proj_w, up_proj_w, down_proj_w):
    B, S, H = hidden_states.shape
    hid = hidden_states.reshape(-1, H)
    N = hid.shape[0]
    E = gate_w.shape[0]
    I = gate_proj_w.shape[1]
    C = int(N * CAPACITY_FACTOR / E)
    C = max(C, 1)
    C = min(C, N)

    def _router_kernel(x_ref, gw_ref, scale_ref, bias_ref,
                       idx_ref, prob_ref, aux_ref):
        x = x_ref[...]
        mean = jnp.mean(x, axis=1, keepdims=True)
        xc = x - mean
        var = jnp.mean(xc * xc, axis=1, keepdims=True)
        xn = xc * jax.lax.rsqrt(var + LN_EPS) * scale_ref[...] + bias_ref[...]
        logits = jax.lax.dot_general(
            gw_ref[...], xn, (((1,), (1,)), ((), ())),
            preferred_element_type=jnp.float32)

        tokmax = jnp.max(logits, axis=1, keepdims=True)
        ex = jnp.exp(logits - tokmax)
        denom = jnp.sum(ex, axis=1, keepdims=True)
        pfull = ex / denom

        expmax = jnp.max(logits, axis=0, keepdims=True)
        lse = jnp.log(jnp.sum(jnp.exp(logits - expmax), axis=0,
                              keepdims=True)) + expmax
        aux_ref[...] = jnp.full((1, 1), 0.001, jnp.float32) * jnp.mean(lse * lse)

        iota_c = jax.lax.broadcasted_iota(jnp.int32, (E, N), 1)
        work = pfull
        idx_cols = []
        val_cols = []
        for _ in range(C):
            cur = jnp.max(work, axis=1, keepdims=True)
            cand = jnp.where(work == cur, iota_c, jnp.int32(N))
            amin = jnp.min(cand, axis=1, keepdims=True)
            hit = iota_c == amin
            work = jnp.where(hit, -1.0, work)
            idx_cols.append(amin)
            val_cols.append(cur)
        idx_ref[...] = jnp.concatenate(idx_cols, axis=1)
        prob_ref[...] = jnp.concatenate(val_cols, axis=1)

    idx, prob, aux = pl.pallas_call(
        _router_kernel,
        out_shape=[
            jax.ShapeDtypeStruct((E, C), jnp.int32),
            jax.ShapeDtypeStruct((E, C), jnp.float32),
            jax.ShapeDtypeStruct((1, 1), jnp.float32),
        ],
    )(hid, gate_w, ln_scale.reshape(1, H), ln_bias.reshape(1, H))

    info = plsc.get_sparse_core_info()
    NW = info.num_cores * info.num_subcores
    rows_per_w = (E * C) // NW
    mesh = plsc.VectorSubcoreMesh(core_axis_name="c", subcore_axis_name="s")

    @functools.partial(
        pl.kernel, mesh=mesh,
        out_type=jax.ShapeDtypeStruct((E * C, H), jnp.float32),
        scratch_types=[
            pltpu.VMEM((rows_per_w,), jnp.int32),
            pltpu.VMEM((rows_per_w, H), jnp.float32),
            pltpu.SemaphoreType.DMA,
        ],
    )
    def _sc_gather(hid_hbm, idxf_hbm, out_hbm, idx_v, rows_v, sem):
        wid = lax.axis_index("s") * info.num_cores + lax.axis_index("c")
        base = wid * rows_per_w
        pltpu.sync_copy(idxf_hbm.at[pl.ds(base, rows_per_w)], idx_v)
        pltpu.async_copy(hid_hbm.at[idx_v], rows_v, sem).wait()
        pltpu.sync_copy(rows_v, out_hbm.at[pl.ds(base, rows_per_w)])

    xgath = _sc_gather(hid, idx.reshape(-1))

    def _ffn_kernel(idx_ref, prob_ref, xin_ref, gp_ref, up_ref, dp_ref,
                    out_ref, cnt_ref):
        e = pl.program_id(0)

        @pl.when(e == 0)
        def _():
            out_ref[...] = jnp.zeros_like(out_ref)
            cnt_ref[...] = jnp.zeros_like(cnt_ref)

        x = xin_ref[...]
        g = jax.lax.dot_general(x, gp_ref[0], (((1,), (1,)), ((), ())),
                                preferred_element_type=jnp.float32)
        u = jax.lax.dot_general(x, up_ref[0], (((1,), (1,)), ((), ())),
                                preferred_element_type=jnp.float32)
        h = g * jax.nn.sigmoid(g) * u
        o = jax.lax.dot_general(h, dp_ref[0], (((1,), (1,)), ((), ())),
                                preferred_element_type=jnp.float32)
        for c in range(C):
            t = idx_ref[e, c]
            p = prob_ref[e, c]
            out_ref[t, :] = out_ref[t, :] + o[c, :] * p
            cnt_ref[pl.ds(t, 1), :] = cnt_ref[pl.ds(t, 1), :] + p

        @pl.when(e == pl.num_programs(0) - 1)
        def _():
            out_ref[...] = out_ref[...] / jnp.maximum(cnt_ref[...], EPS)

    out = pl.pallas_call(
        _ffn_kernel,
        grid=(E,),
        in_specs=[
            pl.BlockSpec(memory_space=pltpu.SMEM),
            pl.BlockSpec(memory_space=pltpu.SMEM),
            pl.BlockSpec((C, H), lambda e: (e, 0)),
            pl.BlockSpec((1, I, H), lambda e: (e, 0, 0)),
            pl.BlockSpec((1, I, H), lambda e: (e, 0, 0)),
            pl.BlockSpec((1, H, I), lambda e: (e, 0, 0)),
        ],
        out_specs=pl.BlockSpec((N, H), lambda e: (0, 0)),
        out_shape=jax.ShapeDtypeStruct((N, H), jnp.float32),
        scratch_shapes=[pltpu.VMEM((N, 1), jnp.float32)],
        compiler_params=pltpu.CompilerParams(
            dimension_semantics=("arbitrary",)),
    )(idx, prob, xgath.reshape(E * C, H), gate_proj_w, up_proj_w, down_proj_w)

    return out.reshape(B, S, H), aux.reshape(())
